# baseline (device time: 62175 ns/iter reference)
import jax
import jax.numpy as jnp
from jax import lax
from jax.experimental import pallas as pl
from jax.experimental.pallas import tpu as pltpu

N_DEV = 16
WINDOW = 128


def kernel(x, Wq, K_ext, V_ext, Wo):
    B, Sq, D = x.shape
    _, Skv, H_loc, Dh = K_ext.shape
    d_loc = H_loc * Dh
    CH = Sq // N_DEV

    def body(x_ref, wq_ref, k_ref, v_ref, wo_ref, out_ref,
             wq_s, wo_s, q_s, ctx_s, acc_s, rs_buf,
             load_sems, rs_send_sems, rs_recv_sems, ag_send_sems,
             ag_recv_sems):
        my = lax.axis_index("i")

        rs_buf[...] = jnp.zeros_like(rs_buf)

        wq_dma = pltpu.make_async_copy(
            wq_ref.at[:, pl.ds(my * d_loc, d_loc)], wq_s, load_sems.at[0])
        wq_dma.start()
        wo_dma = pltpu.make_async_copy(
            wo_ref.at[pl.ds(my * d_loc, d_loc), :], wo_s, load_sems.at[1])
        wo_dma.start()

        bar = pltpu.get_barrier_semaphore()
        for j in range(N_DEV):
            pl.semaphore_signal(bar, inc=1, device_id=(j,),
                                device_id_type=pl.DeviceIdType.MESH)
        pl.semaphore_wait(bar, N_DEV)

        wq_dma.wait()
        wo_dma.wait()

        xb = x_ref[...].astype(jnp.bfloat16).reshape(B * Sq, D)
        q = jnp.dot(xb, wq_s[...].astype(jnp.bfloat16),
                    preferred_element_type=jnp.float32)
        q_s[...] = (q * 0.125).reshape(B, Sq, d_loc).astype(jnp.bfloat16)

        qi = lax.broadcasted_iota(jnp.int32, (Sq, Skv), 0)
        ki = lax.broadcasted_iota(jnp.int32, (Sq, Skv), 1)
        mask = jnp.abs(qi - ki) <= WINDOW

        RB = 128
        bands = []
        for rb in range(Sq // RB):
            c0 = max(0, rb * RB - WINDOW)
            c1 = min(Skv, (rb + 1) * RB + WINDOW)
            bands.append((rb * RB, c0, c1 - c0))

        def round_compute_send(b, rb):
            r0, c0, w = bands[rb]
            for h in range(H_loc):
                qbh = q_s[b, r0:r0 + RB, h * Dh:(h + 1) * Dh]
                kbh = k_ref[b, c0:c0 + w, h, :].astype(jnp.bfloat16)
                vbh = v_ref[b, c0:c0 + w, h, :].astype(jnp.bfloat16)
                s = lax.dot_general(
                    qbh, kbh, (((1,), (1,)), ((), ())),
                    preferred_element_type=jnp.float32)
                e = jnp.where(mask[r0:r0 + RB, c0:c0 + w],
                              jnp.exp(s), 0.0).astype(jnp.bfloat16)
                vaug = jnp.concatenate(
                    [vbh, jnp.ones((w, 1), jnp.bfloat16)], axis=1)
                ctx_aug = jnp.dot(e, vaug,
                                  preferred_element_type=jnp.float32)
                ctx_s[b, r0:r0 + RB, h * Dh:(h + 1) * Dh] = (
                    ctx_aug[:, :Dh] / ctx_aug[:, Dh:Dh + 1]
                ).astype(jnp.bfloat16)

            part = jnp.dot(ctx_s[b, r0:r0 + RB, :],
                           wo_s[...].astype(jnp.bfloat16),
                           preferred_element_type=jnp.float32)
            acc_s[b, r0:r0 + RB, :] = part.astype(jnp.bfloat16)

            for j in range(4 * rb, 4 * rb + 4):
                @pl.when(my != j)
                def _(j=j, b=b):
                    pltpu.make_async_remote_copy(
                        src_ref=acc_s.at[pl.ds(b, 1), pl.ds(j * CH, CH), :],
                        dst_ref=rs_buf.at[pl.ds(b * N_DEV + my, 1)],
                        send_sem=rs_send_sems.at[b],
                        recv_sem=rs_recv_sems.at[b],
                        device_id=(j,),
                        device_id_type=pl.DeviceIdType.MESH,
                    ).start()

        def rs_wait_reduce_ag(b):
            for j in range(N_DEV):
                @pl.when(my != j)
                def _(j=j, b=b):
                    pltpu.make_async_remote_copy(
                        src_ref=acc_s.at[pl.ds(b, 1), pl.ds(j * CH, CH), :],
                        dst_ref=rs_buf.at[pl.ds(b * N_DEV + j, 1)],
                        send_sem=rs_send_sems.at[b],
                        recv_sem=rs_recv_sems.at[b],
                        device_id=(j,),
                        device_id_type=pl.DeviceIdType.MESH,
                    ).wait_recv()

            own = acc_s[b, pl.ds(my * CH, CH), :].astype(jnp.float32)
            red = own + jnp.sum(
                rs_buf[b * N_DEV:(b + 1) * N_DEV].astype(jnp.float32), axis=0)
            out_ref[b, pl.ds(my * CH, CH), :] = red.astype(jnp.bfloat16)

            for j in range(N_DEV):
                @pl.when(my != j)
                def _(j=j, b=b):
                    pltpu.make_async_remote_copy(
                        src_ref=out_ref.at[pl.ds(b, 1), pl.ds(my * CH, CH), :],
                        dst_ref=out_ref.at[pl.ds(b, 1), pl.ds(my * CH, CH), :],
                        send_sem=ag_send_sems.at[b],
                        recv_sem=ag_recv_sems.at[b],
                        device_id=(j,),
                        device_id_type=pl.DeviceIdType.MESH,
                    ).start()

        for rb in range(Sq // RB):
            round_compute_send(0, rb)
        round_compute_send(1, 0)
        rs_wait_reduce_ag(0)
        for rb in range(1, Sq // RB):
            round_compute_send(1, rb)
        rs_wait_reduce_ag(1)

        for b in range(B):
            for j in range(N_DEV):
                @pl.when(my != j)
                def _(j=j, b=b):
                    pltpu.make_async_remote_copy(
                        src_ref=out_ref.at[pl.ds(b, 1), pl.ds(j * CH, CH), :],
                        dst_ref=out_ref.at[pl.ds(b, 1), pl.ds(j * CH, CH), :],
                        send_sem=ag_send_sems.at[b],
                        recv_sem=ag_recv_sems.at[b],
                        device_id=(j,),
                        device_id_type=pl.DeviceIdType.MESH,
                    ).wait_recv()

        for b in range(B):
            for j in range(N_DEV):
                @pl.when(my != j)
                def _(j=j, b=b):
                    pltpu.make_async_remote_copy(
                        src_ref=acc_s.at[pl.ds(b, 1), pl.ds(j * CH, CH), :],
                        dst_ref=rs_buf.at[pl.ds(b * N_DEV + j, 1)],
                        send_sem=rs_send_sems.at[b],
                        recv_sem=rs_recv_sems.at[b],
                        device_id=(j,),
                        device_id_type=pl.DeviceIdType.MESH,
                    ).wait_send()
                    pltpu.make_async_remote_copy(
                        src_ref=out_ref.at[pl.ds(b, 1), pl.ds(my * CH, CH), :],
                        dst_ref=out_ref.at[pl.ds(b, 1), pl.ds(my * CH, CH), :],
                        send_sem=ag_send_sems.at[b],
                        recv_sem=ag_recv_sems.at[b],
                        device_id=(j,),
                        device_id_type=pl.DeviceIdType.MESH,
                    ).wait_send()

    return pl.pallas_call(
        body,
        out_shape=jax.ShapeDtypeStruct((B, Sq, D), jnp.bfloat16),
        in_specs=[
            pl.BlockSpec(memory_space=pltpu.MemorySpace.VMEM),
            pl.BlockSpec(memory_space=pltpu.MemorySpace.HBM),
            pl.BlockSpec(memory_space=pltpu.MemorySpace.VMEM),
            pl.BlockSpec(memory_space=pltpu.MemorySpace.VMEM),
            pl.BlockSpec(memory_space=pltpu.MemorySpace.HBM),
        ],
        out_specs=pl.BlockSpec(memory_space=pltpu.MemorySpace.VMEM),
        scratch_shapes=[
            pltpu.VMEM((D, d_loc), jnp.float32),
            pltpu.VMEM((d_loc, D), jnp.float32),
            pltpu.VMEM((B, Sq, d_loc), jnp.bfloat16),
            pltpu.VMEM((B, Sq, d_loc), jnp.bfloat16),
            pltpu.VMEM((B, Sq, D), jnp.bfloat16),
            pltpu.VMEM((B * N_DEV, CH, D), jnp.bfloat16),
            pltpu.SemaphoreType.DMA((2,)),
            pltpu.SemaphoreType.DMA((2,)),
            pltpu.SemaphoreType.DMA((2,)),
            pltpu.SemaphoreType.DMA((2,)),
            pltpu.SemaphoreType.DMA((2,)),
        ],
        compiler_params=pltpu.CompilerParams(collective_id=0),
    )(x, Wq, K_ext, V_ext, Wo)


# device time: 47601 ns/iter; 1.3062x vs baseline; 1.3062x over previous
import jax
import jax.numpy as jnp
from jax import lax
from jax.experimental import pallas as pl
from jax.experimental.pallas import tpu as pltpu

N_DEV = 16
NP = 4
NZ = 4
WINDOW = 128


def kernel(x, Wq, K_ext, V_ext, Wo):
    B, Sq, D = x.shape
    _, Skv, H_loc, Dh = K_ext.shape
    d_loc = H_loc * Dh
    CH = Sq // N_DEV
    GR = Sq // NP

    def body(x_ref, wq_ref, k_ref, v_ref, wo_ref, out_ref,
             wq_s, wo_s, q_s, ctx_s, acc_s, colgrp_s, pbuf, zbuf,
             load_sems, p_send, p_recv, z_send, z_recv,
             cag_send, cag_recv, pag_send, pag_recv):
        my = lax.axis_index("i")
        z = my // NP
        p = lax.rem(my, NP)

        pbuf[...] = jnp.zeros_like(pbuf)
        zbuf[...] = jnp.zeros_like(zbuf)

        wq_dma = pltpu.make_async_copy(
            wq_ref.at[:, pl.ds(my * d_loc, d_loc)], wq_s, load_sems.at[0])
        wq_dma.start()
        wo_dma = pltpu.make_async_copy(
            wo_ref.at[pl.ds(my * d_loc, d_loc), :], wo_s, load_sems.at[1])
        wo_dma.start()

        bar = pltpu.get_barrier_semaphore()
        for pp in range(NP):
            @pl.when(p != pp)
            def _(pp=pp):
                pl.semaphore_signal(bar, inc=1, device_id=(NP * z + pp,),
                                    device_id_type=pl.DeviceIdType.MESH)
        for zz in range(NZ):
            @pl.when(z != zz)
            def _(zz=zz):
                pl.semaphore_signal(bar, inc=1, device_id=(NP * zz + p,),
                                    device_id_type=pl.DeviceIdType.MESH)
        pl.semaphore_wait(bar, 6)

        wq_dma.wait()
        wo_dma.wait()

        xb = x_ref[...].astype(jnp.bfloat16).reshape(B * Sq, D)
        q = jnp.dot(xb, (wq_s[...] * 0.125).astype(jnp.bfloat16),
                    preferred_element_type=jnp.float32)
        q_s[...] = q.reshape(B, Sq, d_loc).astype(jnp.bfloat16)

        qi = lax.broadcasted_iota(jnp.int32, (Sq, Skv), 0)
        ki = lax.broadcasted_iota(jnp.int32, (Sq, Skv), 1)
        mask = jnp.abs(qi - ki) <= WINDOW
        ones_col = jnp.ones((Skv, 1), jnp.bfloat16)

        def attn(b):
            for h in range(H_loc):
                qbh = q_s[b, :, h * Dh:(h + 1) * Dh]
                kbh = k_ref[b, :, h, :].astype(jnp.bfloat16)
                s = lax.dot_general(
                    qbh, kbh, (((1,), (1,)), ((), ())),
                    preferred_element_type=jnp.float32)
                e = jnp.where(mask, jnp.exp(s), 0.0).astype(jnp.bfloat16)
                vbh = v_ref[b, :, h, :].astype(jnp.bfloat16)
                ctx_aug = jnp.dot(e, jnp.concatenate([vbh, ones_col], axis=1),
                                  preferred_element_type=jnp.float32)
                ctx_s[b, :, h * Dh:(h + 1) * Dh] = (
                    ctx_aug[:, :Dh] / ctx_aug[:, Dh:Dh + 1]
                ).astype(jnp.bfloat16)

        def proj(b):
            part = jnp.dot(ctx_s[b], wo_s[...].astype(jnp.bfloat16),
                           preferred_element_type=jnp.float32)
            acc_s[b] = part.astype(jnp.bfloat16)

        def ph1_send(b):
            for pp in range(NP):
                @pl.when(p != pp)
                def _(pp=pp, b=b):
                    pltpu.make_async_remote_copy(
                        src_ref=acc_s.at[pl.ds(b, 1), pl.ds(pp * GR, GR), :],
                        dst_ref=pbuf.at[pl.ds(b * NP + p, 1)],
                        send_sem=p_send.at[b], recv_sem=p_recv.at[b],
                        device_id=(NP * z + pp,),
                        device_id_type=pl.DeviceIdType.MESH,
                    ).start()

        def ph1_wait_reduce(b):
            for pp in range(NP):
                @pl.when(p != pp)
                def _(pp=pp, b=b):
                    pltpu.make_async_remote_copy(
                        src_ref=acc_s.at[pl.ds(b, 1), pl.ds(pp * GR, GR), :],
                        dst_ref=pbuf.at[pl.ds(b * NP + pp, 1)],
                        send_sem=p_send.at[b], recv_sem=p_recv.at[b],
                        device_id=(NP * z + pp,),
                        device_id_type=pl.DeviceIdType.MESH,
                    ).wait_recv()
            grp = acc_s[b, pl.ds(p * GR, GR), :].astype(jnp.float32)
            grp = grp + jnp.sum(
                pbuf[b * NP:(b + 1) * NP].astype(jnp.float32), axis=0)
            colgrp_s[b] = grp.astype(jnp.bfloat16)

        def ph2_send(b):
            for zz in range(NZ):
                @pl.when(z != zz)
                def _(zz=zz, b=b):
                    pltpu.make_async_remote_copy(
                        src_ref=colgrp_s.at[pl.ds(b, 1), pl.ds(zz * CH, CH), :],
                        dst_ref=zbuf.at[pl.ds(b * NZ + z, 1)],
                        send_sem=z_send.at[b], recv_sem=z_recv.at[b],
                        device_id=(NP * zz + p,),
                        device_id_type=pl.DeviceIdType.MESH,
                    ).start()

        def ph2_wait_reduce(b):
            for zz in range(NZ):
                @pl.when(z != zz)
                def _(zz=zz, b=b):
                    pltpu.make_async_remote_copy(
                        src_ref=colgrp_s.at[pl.ds(b, 1), pl.ds(zz * CH, CH), :],
                        dst_ref=zbuf.at[pl.ds(b * NZ + zz, 1)],
                        send_sem=z_send.at[b], recv_sem=z_recv.at[b],
                        device_id=(NP * zz + p,),
                        device_id_type=pl.DeviceIdType.MESH,
                    ).wait_recv()
            red = colgrp_s[b, pl.ds(z * CH, CH), :].astype(jnp.float32)
            red = red + jnp.sum(
                zbuf[b * NZ:(b + 1) * NZ].astype(jnp.float32), axis=0)
            out_ref[b, pl.ds(p * GR + z * CH, CH), :] = red.astype(jnp.bfloat16)

        def ph3_send(b):
            for zz in range(NZ):
                @pl.when(z != zz)
                def _(zz=zz, b=b):
                    pltpu.make_async_remote_copy(
                        src_ref=out_ref.at[
                            pl.ds(b, 1), pl.ds(p * GR + z * CH, CH), :],
                        dst_ref=out_ref.at[
                            pl.ds(b, 1), pl.ds(p * GR + z * CH, CH), :],
                        send_sem=cag_send.at[b], recv_sem=cag_recv.at[b],
                        device_id=(NP * zz + p,),
                        device_id_type=pl.DeviceIdType.MESH,
                    ).start()

        def ph3_wait(b):
            for zz in range(NZ):
                @pl.when(z != zz)
                def _(zz=zz, b=b):
                    pltpu.make_async_remote_copy(
                        src_ref=out_ref.at[
                            pl.ds(b, 1), pl.ds(p * GR + zz * CH, CH), :],
                        dst_ref=out_ref.at[
                            pl.ds(b, 1), pl.ds(p * GR + zz * CH, CH), :],
                        send_sem=cag_send.at[b], recv_sem=cag_recv.at[b],
                        device_id=(NP * zz + p,),
                        device_id_type=pl.DeviceIdType.MESH,
                    ).wait_recv()

        def ph4_send(b):
            for pp in range(NP):
                @pl.when(p != pp)
                def _(pp=pp, b=b):
                    pltpu.make_async_remote_copy(
                        src_ref=out_ref.at[pl.ds(b, 1), pl.ds(p * GR, GR), :],
                        dst_ref=out_ref.at[pl.ds(b, 1), pl.ds(p * GR, GR), :],
                        send_sem=pag_send.at[b], recv_sem=pag_recv.at[b],
                        device_id=(NP * z + pp,),
                        device_id_type=pl.DeviceIdType.MESH,
                    ).start()

        def ph4_wait(b):
            for pp in range(NP):
                @pl.when(p != pp)
                def _(pp=pp, b=b):
                    pltpu.make_async_remote_copy(
                        src_ref=out_ref.at[pl.ds(b, 1), pl.ds(pp * GR, GR), :],
                        dst_ref=out_ref.at[pl.ds(b, 1), pl.ds(pp * GR, GR), :],
                        send_sem=pag_send.at[b], recv_sem=pag_recv.at[b],
                        device_id=(NP * z + pp,),
                        device_id_type=pl.DeviceIdType.MESH,
                    ).wait_recv()

        attn(0)
        proj(0)
        ph1_send(0)
        attn(1)
        ph1_wait_reduce(0)
        ph2_send(0)
        proj(1)
        ph1_send(1)
        ph2_wait_reduce(0)
        ph3_send(0)
        ph1_wait_reduce(1)
        ph2_send(1)
        ph3_wait(0)
        ph4_send(0)
        ph2_wait_reduce(1)
        ph3_send(1)
        ph4_wait(0)
        ph3_wait(1)
        ph4_send(1)
        ph4_wait(1)

        for b in range(B):
            for pp in range(NP):
                @pl.when(p != pp)
                def _(pp=pp, b=b):
                    pltpu.make_async_remote_copy(
                        src_ref=acc_s.at[pl.ds(b, 1), pl.ds(pp * GR, GR), :],
                        dst_ref=pbuf.at[pl.ds(b * NP + pp, 1)],
                        send_sem=p_send.at[b], recv_sem=p_recv.at[b],
                        device_id=(NP * z + pp,),
                        device_id_type=pl.DeviceIdType.MESH,
                    ).wait_send()
                    pltpu.make_async_remote_copy(
                        src_ref=out_ref.at[pl.ds(b, 1), pl.ds(p * GR, GR), :],
                        dst_ref=out_ref.at[pl.ds(b, 1), pl.ds(p * GR, GR), :],
                        send_sem=pag_send.at[b], recv_sem=pag_recv.at[b],
                        device_id=(NP * z + pp,),
                        device_id_type=pl.DeviceIdType.MESH,
                    ).wait_send()
            for zz in range(NZ):
                @pl.when(z != zz)
                def _(zz=zz, b=b):
                    pltpu.make_async_remote_copy(
                        src_ref=colgrp_s.at[pl.ds(b, 1), pl.ds(zz * CH, CH), :],
                        dst_ref=zbuf.at[pl.ds(b * NZ + zz, 1)],
                        send_sem=z_send.at[b], recv_sem=z_recv.at[b],
                        device_id=(NP * zz + p,),
                        device_id_type=pl.DeviceIdType.MESH,
                    ).wait_send()
                    pltpu.make_async_remote_copy(
                        src_ref=out_ref.at[
                            pl.ds(b, 1), pl.ds(p * GR + z * CH, CH), :],
                        dst_ref=out_ref.at[
                            pl.ds(b, 1), pl.ds(p * GR + z * CH, CH), :],
                        send_sem=cag_send.at[b], recv_sem=cag_recv.at[b],
                        device_id=(NP * zz + p,),
                        device_id_type=pl.DeviceIdType.MESH,
                    ).wait_send()

    return pl.pallas_call(
        body,
        out_shape=jax.ShapeDtypeStruct((B, Sq, D), jnp.bfloat16),
        in_specs=[
            pl.BlockSpec(memory_space=pltpu.MemorySpace.VMEM),
            pl.BlockSpec(memory_space=pltpu.MemorySpace.HBM),
            pl.BlockSpec(memory_space=pltpu.MemorySpace.VMEM),
            pl.BlockSpec(memory_space=pltpu.MemorySpace.VMEM),
            pl.BlockSpec(memory_space=pltpu.MemorySpace.HBM),
        ],
        out_specs=pl.BlockSpec(memory_space=pltpu.MemorySpace.VMEM),
        scratch_shapes=[
            pltpu.VMEM((D, d_loc), jnp.float32),
            pltpu.VMEM((d_loc, D), jnp.float32),
            pltpu.VMEM((B, Sq, d_loc), jnp.bfloat16),
            pltpu.VMEM((B, Sq, d_loc), jnp.bfloat16),
            pltpu.VMEM((B, Sq, D), jnp.bfloat16),
            pltpu.VMEM((B, GR, D), jnp.bfloat16),
            pltpu.VMEM((B * NP, GR, D), jnp.bfloat16),
            pltpu.VMEM((B * NZ, CH, D), jnp.bfloat16),
            pltpu.SemaphoreType.DMA((2,)),
            pltpu.SemaphoreType.DMA((2,)),
            pltpu.SemaphoreType.DMA((2,)),
            pltpu.SemaphoreType.DMA((2,)),
            pltpu.SemaphoreType.DMA((2,)),
            pltpu.SemaphoreType.DMA((2,)),
            pltpu.SemaphoreType.DMA((2,)),
            pltpu.SemaphoreType.DMA((2,)),
            pltpu.SemaphoreType.DMA((2,)),
        ],
        compiler_params=pltpu.CompilerParams(collective_id=0),
    )(x, Wq, K_ext, V_ext, Wo)


# device time: 43102 ns/iter; 1.4425x vs baseline; 1.1044x over previous
import jax
import jax.numpy as jnp
from jax import lax
from jax.experimental import pallas as pl
from jax.experimental.pallas import tpu as pltpu

N_DEV = 16
NP = 4
NZ = 4
WINDOW = 128


def kernel(x, Wq, K_ext, V_ext, Wo):
    B, Sq, D = x.shape
    _, Skv, H_loc, Dh = K_ext.shape
    d_loc = H_loc * Dh
    CH = Sq // N_DEV
    GR = Sq // NP

    def body(x_ref, wq_ref, k_ref, v_ref, wo_ref, out_ref,
             wq_s, wo_s, q_s, ctx_s, acc_s, colgrp_s, pbuf, zbuf,
             kt_s, vp_s,
             load_sems, p_send, p_recv, z_send, z_recv,
             cag_send, cag_recv, pag_send, pag_recv):
        my = lax.axis_index("i")
        z = my // NP
        p = lax.rem(my, NP)

        pbuf[...] = jnp.zeros_like(pbuf)
        zbuf[...] = jnp.zeros_like(zbuf)

        wq_dma = pltpu.make_async_copy(
            wq_ref.at[:, pl.ds(my * d_loc, d_loc)], wq_s, load_sems.at[0])
        wq_dma.start()
        wo_dma = pltpu.make_async_copy(
            wo_ref.at[pl.ds(my * d_loc, d_loc), :], wo_s, load_sems.at[1])
        wo_dma.start()

        bar = pltpu.get_barrier_semaphore()
        for pp in range(NP):
            @pl.when(p != pp)
            def _(pp=pp):
                pl.semaphore_signal(bar, inc=1, device_id=(NP * z + pp,),
                                    device_id_type=pl.DeviceIdType.MESH)
        for zz in range(NZ):
            @pl.when(z != zz)
            def _(zz=zz):
                pl.semaphore_signal(bar, inc=1, device_id=(NP * zz + p,),
                                    device_id_type=pl.DeviceIdType.MESH)
        pl.semaphore_wait(bar, 6)

        wq_dma.wait()
        wo_dma.wait()

        xb = x_ref[...].astype(jnp.bfloat16).reshape(B * Sq, D)
        q = jnp.dot(xb, (wq_s[...] * 0.125).astype(jnp.bfloat16),
                    preferred_element_type=jnp.float32)
        q_s[...] = q.reshape(B, Sq, d_loc).astype(jnp.bfloat16)

        qi = lax.broadcasted_iota(jnp.int32, (Sq, Skv), 0)
        ki = lax.broadcasted_iota(jnp.int32, (Sq, Skv), 1)
        mask = jnp.abs(qi - ki) <= WINDOW
        ones_col = jnp.ones((Skv, 1), jnp.bfloat16)

        def attn(b):
            kt_s[...] = jnp.reshape(
                k_ref[b][...], (Skv, d_loc)).astype(jnp.bfloat16).T
            vp_s[...] = jnp.reshape(
                v_ref[b][...], (Skv, d_loc)).astype(jnp.bfloat16)
            for h in range(H_loc):
                qbh = q_s[b, :, h * Dh:(h + 1) * Dh]
                s = jnp.dot(qbh, kt_s[h * Dh:(h + 1) * Dh, :],
                            preferred_element_type=jnp.float32)
                e = jnp.where(mask, jnp.exp(s), 0.0).astype(jnp.bfloat16)
                vbh = vp_s[:, h * Dh:(h + 1) * Dh]
                ctx_aug = jnp.dot(e, jnp.concatenate([vbh, ones_col], axis=1),
                                  preferred_element_type=jnp.float32)
                ctx_s[b, :, h * Dh:(h + 1) * Dh] = (
                    ctx_aug[:, :Dh] / ctx_aug[:, Dh:Dh + 1]
                ).astype(jnp.bfloat16)

        def proj(b):
            part = jnp.dot(ctx_s[b], wo_s[...].astype(jnp.bfloat16),
                           preferred_element_type=jnp.float32)
            acc_s[b] = part.astype(jnp.bfloat16)

        def ph1_send(b):
            for pp in range(NP):
                @pl.when(p != pp)
                def _(pp=pp, b=b):
                    pltpu.make_async_remote_copy(
                        src_ref=acc_s.at[pl.ds(b, 1), pl.ds(pp * GR, GR), :],
                        dst_ref=pbuf.at[pl.ds(b * NP + p, 1)],
                        send_sem=p_send.at[b], recv_sem=p_recv.at[b],
                        device_id=(NP * z + pp,),
                        device_id_type=pl.DeviceIdType.MESH,
                    ).start()

        def ph1_wait_reduce(b):
            for pp in range(NP):
                @pl.when(p != pp)
                def _(pp=pp, b=b):
                    pltpu.make_async_remote_copy(
                        src_ref=acc_s.at[pl.ds(b, 1), pl.ds(pp * GR, GR), :],
                        dst_ref=pbuf.at[pl.ds(b * NP + pp, 1)],
                        send_sem=p_send.at[b], recv_sem=p_recv.at[b],
                        device_id=(NP * z + pp,),
                        device_id_type=pl.DeviceIdType.MESH,
                    ).wait_recv()
            grp = acc_s[b, pl.ds(p * GR, GR), :].astype(jnp.float32)
            grp = grp + jnp.sum(
                pbuf[b * NP:(b + 1) * NP].astype(jnp.float32), axis=0)
            colgrp_s[b] = grp.astype(jnp.bfloat16)

        def ph2_send(b):
            for zz in range(NZ):
                @pl.when(z != zz)
                def _(zz=zz, b=b):
                    pltpu.make_async_remote_copy(
                        src_ref=colgrp_s.at[pl.ds(b, 1), pl.ds(zz * CH, CH), :],
                        dst_ref=zbuf.at[pl.ds(b * NZ + z, 1)],
                        send_sem=z_send.at[b], recv_sem=z_recv.at[b],
                        device_id=(NP * zz + p,),
                        device_id_type=pl.DeviceIdType.MESH,
                    ).start()

        def ph2_wait_reduce(b):
            for zz in range(NZ):
                @pl.when(z != zz)
                def _(zz=zz, b=b):
                    pltpu.make_async_remote_copy(
                        src_ref=colgrp_s.at[pl.ds(b, 1), pl.ds(zz * CH, CH), :],
                        dst_ref=zbuf.at[pl.ds(b * NZ + zz, 1)],
                        send_sem=z_send.at[b], recv_sem=z_recv.at[b],
                        device_id=(NP * zz + p,),
                        device_id_type=pl.DeviceIdType.MESH,
                    ).wait_recv()
            red = colgrp_s[b, pl.ds(z * CH, CH), :].astype(jnp.float32)
            red = red + jnp.sum(
                zbuf[b * NZ:(b + 1) * NZ].astype(jnp.float32), axis=0)
            out_ref[b, pl.ds(p * GR + z * CH, CH), :] = red.astype(jnp.bfloat16)

        def ph3_send(b):
            for zz in range(NZ):
                @pl.when(z != zz)
                def _(zz=zz, b=b):
                    pltpu.make_async_remote_copy(
                        src_ref=out_ref.at[
                            pl.ds(b, 1), pl.ds(p * GR + z * CH, CH), :],
                        dst_ref=out_ref.at[
                            pl.ds(b, 1), pl.ds(p * GR + z * CH, CH), :],
                        send_sem=cag_send.at[b], recv_sem=cag_recv.at[b],
                        device_id=(NP * zz + p,),
                        device_id_type=pl.DeviceIdType.MESH,
                    ).start()

        def ph3_wait(b):
            for zz in range(NZ):
                @pl.when(z != zz)
                def _(zz=zz, b=b):
                    pltpu.make_async_remote_copy(
                        src_ref=out_ref.at[
                            pl.ds(b, 1), pl.ds(p * GR + zz * CH, CH), :],
                        dst_ref=out_ref.at[
                            pl.ds(b, 1), pl.ds(p * GR + zz * CH, CH), :],
                        send_sem=cag_send.at[b], recv_sem=cag_recv.at[b],
                        device_id=(NP * zz + p,),
                        device_id_type=pl.DeviceIdType.MESH,
                    ).wait_recv()

        def ph4_send(b):
            for pp in range(NP):
                @pl.when(p != pp)
                def _(pp=pp, b=b):
                    pltpu.make_async_remote_copy(
                        src_ref=out_ref.at[pl.ds(b, 1), pl.ds(p * GR, GR), :],
                        dst_ref=out_ref.at[pl.ds(b, 1), pl.ds(p * GR, GR), :],
                        send_sem=pag_send.at[b], recv_sem=pag_recv.at[b],
                        device_id=(NP * z + pp,),
                        device_id_type=pl.DeviceIdType.MESH,
                    ).start()

        def ph4_wait(b):
            for pp in range(NP):
                @pl.when(p != pp)
                def _(pp=pp, b=b):
                    pltpu.make_async_remote_copy(
                        src_ref=out_ref.at[pl.ds(b, 1), pl.ds(pp * GR, GR), :],
                        dst_ref=out_ref.at[pl.ds(b, 1), pl.ds(pp * GR, GR), :],
                        send_sem=pag_send.at[b], recv_sem=pag_recv.at[b],
                        device_id=(NP * z + pp,),
                        device_id_type=pl.DeviceIdType.MESH,
                    ).wait_recv()

        attn(0)
        proj(0)
        ph1_send(0)
        attn(1)
        ph1_wait_reduce(0)
        ph2_send(0)
        proj(1)
        ph1_send(1)
        ph2_wait_reduce(0)
        ph3_send(0)
        ph1_wait_reduce(1)
        ph2_send(1)
        ph3_wait(0)
        ph4_send(0)
        ph2_wait_reduce(1)
        ph3_send(1)
        ph4_wait(0)
        ph3_wait(1)
        ph4_send(1)
        ph4_wait(1)

        for b in range(B):
            for pp in range(NP):
                @pl.when(p != pp)
                def _(pp=pp, b=b):
                    pltpu.make_async_remote_copy(
                        src_ref=acc_s.at[pl.ds(b, 1), pl.ds(pp * GR, GR), :],
                        dst_ref=pbuf.at[pl.ds(b * NP + pp, 1)],
                        send_sem=p_send.at[b], recv_sem=p_recv.at[b],
                        device_id=(NP * z + pp,),
                        device_id_type=pl.DeviceIdType.MESH,
                    ).wait_send()
                    pltpu.make_async_remote_copy(
                        src_ref=out_ref.at[pl.ds(b, 1), pl.ds(p * GR, GR), :],
                        dst_ref=out_ref.at[pl.ds(b, 1), pl.ds(p * GR, GR), :],
                        send_sem=pag_send.at[b], recv_sem=pag_recv.at[b],
                        device_id=(NP * z + pp,),
                        device_id_type=pl.DeviceIdType.MESH,
                    ).wait_send()
            for zz in range(NZ):
                @pl.when(z != zz)
                def _(zz=zz, b=b):
                    pltpu.make_async_remote_copy(
                        src_ref=colgrp_s.at[pl.ds(b, 1), pl.ds(zz * CH, CH), :],
                        dst_ref=zbuf.at[pl.ds(b * NZ + zz, 1)],
                        send_sem=z_send.at[b], recv_sem=z_recv.at[b],
                        device_id=(NP * zz + p,),
                        device_id_type=pl.DeviceIdType.MESH,
                    ).wait_send()
                    pltpu.make_async_remote_copy(
                        src_ref=out_ref.at[
                            pl.ds(b, 1), pl.ds(p * GR + z * CH, CH), :],
                        dst_ref=out_ref.at[
                            pl.ds(b, 1), pl.ds(p * GR + z * CH, CH), :],
                        send_sem=cag_send.at[b], recv_sem=cag_recv.at[b],
                        device_id=(NP * zz + p,),
                        device_id_type=pl.DeviceIdType.MESH,
                    ).wait_send()

    return pl.pallas_call(
        body,
        out_shape=jax.ShapeDtypeStruct((B, Sq, D), jnp.bfloat16),
        in_specs=[
            pl.BlockSpec(memory_space=pltpu.MemorySpace.VMEM),
            pl.BlockSpec(memory_space=pltpu.MemorySpace.HBM),
            pl.BlockSpec(memory_space=pltpu.MemorySpace.VMEM),
            pl.BlockSpec(memory_space=pltpu.MemorySpace.VMEM),
            pl.BlockSpec(memory_space=pltpu.MemorySpace.HBM),
        ],
        out_specs=pl.BlockSpec(memory_space=pltpu.MemorySpace.VMEM),
        scratch_shapes=[
            pltpu.VMEM((D, d_loc), jnp.float32),
            pltpu.VMEM((d_loc, D), jnp.float32),
            pltpu.VMEM((B, Sq, d_loc), jnp.bfloat16),
            pltpu.VMEM((B, Sq, d_loc), jnp.bfloat16),
            pltpu.VMEM((B, Sq, D), jnp.bfloat16),
            pltpu.VMEM((B, GR, D), jnp.bfloat16),
            pltpu.VMEM((B * NP, GR, D), jnp.bfloat16),
            pltpu.VMEM((B * NZ, CH, D), jnp.bfloat16),
            pltpu.VMEM((d_loc, Skv), jnp.bfloat16),
            pltpu.VMEM((Skv, d_loc), jnp.bfloat16),
            pltpu.SemaphoreType.DMA((2,)),
            pltpu.SemaphoreType.DMA((2,)),
            pltpu.SemaphoreType.DMA((2,)),
            pltpu.SemaphoreType.DMA((2,)),
            pltpu.SemaphoreType.DMA((2,)),
            pltpu.SemaphoreType.DMA((2,)),
            pltpu.SemaphoreType.DMA((2,)),
            pltpu.SemaphoreType.DMA((2,)),
            pltpu.SemaphoreType.DMA((2,)),
        ],
        compiler_params=pltpu.CompilerParams(collective_id=0),
    )(x, Wq, K_ext, V_ext, Wo)


# device time: 40730 ns/iter; 1.5265x vs baseline; 1.0582x over previous
import jax
import jax.numpy as jnp
from jax import lax
from jax.experimental import pallas as pl
from jax.experimental.pallas import tpu as pltpu

N_DEV = 16
NP = 4
NZ = 4
WINDOW = 128


def kernel(x, Wq, K_ext, V_ext, Wo):
    B, Sq, D = x.shape
    _, Skv, H_loc, Dh = K_ext.shape
    d_loc = H_loc * Dh
    CH = Sq // N_DEV
    GR = Sq // NP

    def body(x_ref, wq_ref, k_ref, v_ref, wo_ref, out_ref,
             wq_s, wo_s, q_s, ctx_s, acc_s, colgrp_s, pbuf, zbuf,
             kt_s, vp_s,
             load_sems, p_send, p_recv, z_send, z_recv,
             cag_send, cag_recv, pag_send, pag_recv):
        my = lax.axis_index("i")
        z = my // NP
        p = lax.rem(my, NP)

        pbuf[...] = jnp.zeros_like(pbuf)
        zbuf[...] = jnp.zeros_like(zbuf)

        wq_dma = pltpu.make_async_copy(
            wq_ref.at[:, pl.ds(my * d_loc, d_loc)], wq_s, load_sems.at[0])
        wq_dma.start()
        wo_dma = pltpu.make_async_copy(
            wo_ref.at[pl.ds(my * d_loc, d_loc), :], wo_s, load_sems.at[1])
        wo_dma.start()

        bar = pltpu.get_barrier_semaphore()
        for pp in range(NP):
            @pl.when(p != pp)
            def _(pp=pp):
                pl.semaphore_signal(bar, inc=1, device_id=(NP * z + pp,),
                                    device_id_type=pl.DeviceIdType.MESH)
        for zz in range(NZ):
            @pl.when(z != zz)
            def _(zz=zz):
                pl.semaphore_signal(bar, inc=1, device_id=(NP * zz + p,),
                                    device_id_type=pl.DeviceIdType.MESH)
        pl.semaphore_wait(bar, 6)

        wq_dma.wait()
        wo_dma.wait()

        xb = x_ref[...].astype(jnp.bfloat16).reshape(B * Sq, D)
        q = jnp.dot(xb, (wq_s[...] * 0.125).astype(jnp.bfloat16),
                    preferred_element_type=jnp.float32)
        q_s[...] = q.reshape(B, Sq, d_loc).astype(jnp.bfloat16)

        qi = lax.broadcasted_iota(jnp.int32, (Sq, Skv), 0)
        ki = lax.broadcasted_iota(jnp.int32, (Sq, Skv), 1)
        mask = jnp.abs(qi - ki) <= WINDOW
        ones_col = jnp.ones((Skv, 1), jnp.bfloat16)

        def attn(b):
            kt_s[...] = jnp.reshape(
                k_ref[b][...], (Skv, d_loc)).astype(jnp.bfloat16).T
            vp_s[...] = jnp.reshape(
                v_ref[b][...], (Skv, d_loc)).astype(jnp.bfloat16)
            for h in range(H_loc):
                qbh = q_s[b, :, h * Dh:(h + 1) * Dh]
                s = jnp.dot(qbh, kt_s[h * Dh:(h + 1) * Dh, :],
                            preferred_element_type=jnp.float32)
                e = jnp.where(mask, jnp.exp(s), 0.0).astype(jnp.bfloat16)
                vbh = vp_s[:, h * Dh:(h + 1) * Dh]
                ctx_aug = jnp.dot(e, jnp.concatenate([vbh, ones_col], axis=1),
                                  preferred_element_type=jnp.float32)
                ctx_s[b, :, h * Dh:(h + 1) * Dh] = (
                    ctx_aug[:, :Dh] / ctx_aug[:, Dh:Dh + 1]
                ).astype(jnp.bfloat16)

        HF = GR // 2
        ZH = CH // 2

        def proj_send(b):
            for pp in range(NP):
                part = jnp.dot(ctx_s[b, pp * GR:(pp + 1) * GR, :],
                               wo_s[...].astype(jnp.bfloat16),
                               preferred_element_type=jnp.float32)
                acc_s[b, pp * GR:(pp + 1) * GR, :] = part.astype(jnp.bfloat16)
                for hf in range(2):
                    r = b * 2 + hf

                    @pl.when(p != pp)
                    def _(pp=pp, hf=hf, r=r, b=b):
                        pltpu.make_async_remote_copy(
                            src_ref=acc_s.at[
                                pl.ds(b, 1), pl.ds(pp * GR + hf * HF, HF), :],
                            dst_ref=pbuf.at[pl.ds(r * NP + p, 1)],
                            send_sem=p_send.at[r], recv_sem=p_recv.at[r],
                            device_id=(NP * z + pp,),
                            device_id_type=pl.DeviceIdType.MESH,
                        ).start()

        def ph1_wait_reduce(r):
            b, hf = r // 2, r % 2
            for pp in range(NP):
                @pl.when(p != pp)
                def _(pp=pp, r=r, b=b, hf=hf):
                    pltpu.make_async_remote_copy(
                        src_ref=acc_s.at[
                            pl.ds(b, 1), pl.ds(pp * GR + hf * HF, HF), :],
                        dst_ref=pbuf.at[pl.ds(r * NP + pp, 1)],
                        send_sem=p_send.at[r], recv_sem=p_recv.at[r],
                        device_id=(NP * z + pp,),
                        device_id_type=pl.DeviceIdType.MESH,
                    ).wait_recv()
            grp = acc_s[b, pl.ds(p * GR + hf * HF, HF), :].astype(jnp.float32)
            grp = grp + jnp.sum(
                pbuf[r * NP:(r + 1) * NP].astype(jnp.float32), axis=0)
            colgrp_s[r] = grp.astype(jnp.bfloat16)

        def ph2_send(r):
            for zz in range(NZ):
                @pl.when(z != zz)
                def _(zz=zz, r=r):
                    pltpu.make_async_remote_copy(
                        src_ref=colgrp_s.at[pl.ds(r, 1), pl.ds(zz * ZH, ZH), :],
                        dst_ref=zbuf.at[pl.ds(r * NZ + z, 1)],
                        send_sem=z_send.at[r], recv_sem=z_recv.at[r],
                        device_id=(NP * zz + p,),
                        device_id_type=pl.DeviceIdType.MESH,
                    ).start()

        def my_rows(r):
            b, hf = r // 2, r % 2
            return p * GR + hf * HF + z * ZH

        def ph2_wait_reduce(r):
            b = r // 2
            for zz in range(NZ):
                @pl.when(z != zz)
                def _(zz=zz, r=r):
                    pltpu.make_async_remote_copy(
                        src_ref=colgrp_s.at[pl.ds(r, 1), pl.ds(zz * ZH, ZH), :],
                        dst_ref=zbuf.at[pl.ds(r * NZ + zz, 1)],
                        send_sem=z_send.at[r], recv_sem=z_recv.at[r],
                        device_id=(NP * zz + p,),
                        device_id_type=pl.DeviceIdType.MESH,
                    ).wait_recv()
            red = colgrp_s[r, pl.ds(z * ZH, ZH), :].astype(jnp.float32)
            red = red + jnp.sum(
                zbuf[r * NZ:(r + 1) * NZ].astype(jnp.float32), axis=0)
            out_ref[b, pl.ds(my_rows(r), ZH), :] = red.astype(jnp.bfloat16)

        def ph3_send(r):
            b = r // 2
            for zz in range(NZ):
                @pl.when(z != zz)
                def _(zz=zz, r=r, b=b):
                    pltpu.make_async_remote_copy(
                        src_ref=out_ref.at[pl.ds(b, 1), pl.ds(my_rows(r), ZH), :],
                        dst_ref=out_ref.at[pl.ds(b, 1), pl.ds(my_rows(r), ZH), :],
                        send_sem=cag_send.at[r], recv_sem=cag_recv.at[r],
                        device_id=(NP * zz + p,),
                        device_id_type=pl.DeviceIdType.MESH,
                    ).start()

        def ph3_wait(r):
            b, hf = r // 2, r % 2
            for zz in range(NZ):
                @pl.when(z != zz)
                def _(zz=zz, r=r, b=b, hf=hf):
                    rows = p * GR + hf * HF + zz * ZH
                    pltpu.make_async_remote_copy(
                        src_ref=out_ref.at[pl.ds(b, 1), pl.ds(rows, ZH), :],
                        dst_ref=out_ref.at[pl.ds(b, 1), pl.ds(rows, ZH), :],
                        send_sem=cag_send.at[r], recv_sem=cag_recv.at[r],
                        device_id=(NP * zz + p,),
                        device_id_type=pl.DeviceIdType.MESH,
                    ).wait_recv()

        def ph4_send(r):
            b, hf = r // 2, r % 2
            for pp in range(NP):
                @pl.when(p != pp)
                def _(pp=pp, r=r, b=b, hf=hf):
                    rows = p * GR + hf * HF
                    pltpu.make_async_remote_copy(
                        src_ref=out_ref.at[pl.ds(b, 1), pl.ds(rows, HF), :],
                        dst_ref=out_ref.at[pl.ds(b, 1), pl.ds(rows, HF), :],
                        send_sem=pag_send.at[r], recv_sem=pag_recv.at[r],
                        device_id=(NP * z + pp,),
                        device_id_type=pl.DeviceIdType.MESH,
                    ).start()

        def ph4_wait(r):
            b, hf = r // 2, r % 2
            for pp in range(NP):
                @pl.when(p != pp)
                def _(pp=pp, r=r, b=b, hf=hf):
                    rows = pp * GR + hf * HF
                    pltpu.make_async_remote_copy(
                        src_ref=out_ref.at[pl.ds(b, 1), pl.ds(rows, HF), :],
                        dst_ref=out_ref.at[pl.ds(b, 1), pl.ds(rows, HF), :],
                        send_sem=pag_send.at[r], recv_sem=pag_recv.at[r],
                        device_id=(NP * z + pp,),
                        device_id_type=pl.DeviceIdType.MESH,
                    ).wait_recv()

        attn(0)
        proj_send(0)
        attn(1)
        ph1_wait_reduce(0); ph2_send(0)
        ph1_wait_reduce(1); ph2_send(1)
        proj_send(1)
        ph2_wait_reduce(0); ph3_send(0)
        ph2_wait_reduce(1); ph3_send(1)
        ph1_wait_reduce(2); ph2_send(2)
        ph3_wait(0); ph4_send(0)
        ph1_wait_reduce(3); ph2_send(3)
        ph3_wait(1); ph4_send(1)
        ph2_wait_reduce(2); ph3_send(2)
        ph2_wait_reduce(3); ph3_send(3)
        ph3_wait(2); ph4_send(2)
        ph3_wait(3); ph4_send(3)
        ph4_wait(0); ph4_wait(1); ph4_wait(2); ph4_wait(3)

        for r in range(2 * B):
            b, hf = r // 2, r % 2
            for pp in range(NP):
                @pl.when(p != pp)
                def _(pp=pp, r=r, b=b, hf=hf):
                    pltpu.make_async_remote_copy(
                        src_ref=acc_s.at[
                            pl.ds(b, 1), pl.ds(pp * GR + hf * HF, HF), :],
                        dst_ref=pbuf.at[pl.ds(r * NP + pp, 1)],
                        send_sem=p_send.at[r], recv_sem=p_recv.at[r],
                        device_id=(NP * z + pp,),
                        device_id_type=pl.DeviceIdType.MESH,
                    ).wait_send()
                    rows = p * GR + hf * HF
                    pltpu.make_async_remote_copy(
                        src_ref=out_ref.at[pl.ds(b, 1), pl.ds(rows, HF), :],
                        dst_ref=out_ref.at[pl.ds(b, 1), pl.ds(rows, HF), :],
                        send_sem=pag_send.at[r], recv_sem=pag_recv.at[r],
                        device_id=(NP * z + pp,),
                        device_id_type=pl.DeviceIdType.MESH,
                    ).wait_send()
            for zz in range(NZ):
                @pl.when(z != zz)
                def _(zz=zz, r=r, b=b):
                    pltpu.make_async_remote_copy(
                        src_ref=colgrp_s.at[pl.ds(r, 1), pl.ds(zz * ZH, ZH), :],
                        dst_ref=zbuf.at[pl.ds(r * NZ + zz, 1)],
                        send_sem=z_send.at[r], recv_sem=z_recv.at[r],
                        device_id=(NP * zz + p,),
                        device_id_type=pl.DeviceIdType.MESH,
                    ).wait_send()
                    pltpu.make_async_remote_copy(
                        src_ref=out_ref.at[pl.ds(b, 1), pl.ds(my_rows(r), ZH), :],
                        dst_ref=out_ref.at[pl.ds(b, 1), pl.ds(my_rows(r), ZH), :],
                        send_sem=cag_send.at[r], recv_sem=cag_recv.at[r],
                        device_id=(NP * zz + p,),
                        device_id_type=pl.DeviceIdType.MESH,
                    ).wait_send()

    return pl.pallas_call(
        body,
        out_shape=jax.ShapeDtypeStruct((B, Sq, D), jnp.bfloat16),
        in_specs=[
            pl.BlockSpec(memory_space=pltpu.MemorySpace.VMEM),
            pl.BlockSpec(memory_space=pltpu.MemorySpace.HBM),
            pl.BlockSpec(memory_space=pltpu.MemorySpace.VMEM),
            pl.BlockSpec(memory_space=pltpu.MemorySpace.VMEM),
            pl.BlockSpec(memory_space=pltpu.MemorySpace.HBM),
        ],
        out_specs=pl.BlockSpec(memory_space=pltpu.MemorySpace.VMEM),
        scratch_shapes=[
            pltpu.VMEM((D, d_loc), jnp.float32),
            pltpu.VMEM((d_loc, D), jnp.float32),
            pltpu.VMEM((B, Sq, d_loc), jnp.bfloat16),
            pltpu.VMEM((B, Sq, d_loc), jnp.bfloat16),
            pltpu.VMEM((B, Sq, D), jnp.bfloat16),
            pltpu.VMEM((2 * B, GR // 2, D), jnp.bfloat16),
            pltpu.VMEM((2 * B * NP, GR // 2, D), jnp.bfloat16),
            pltpu.VMEM((2 * B * NZ, CH // 2, D), jnp.bfloat16),
            pltpu.VMEM((d_loc, Skv), jnp.bfloat16),
            pltpu.VMEM((Skv, d_loc), jnp.bfloat16),
            pltpu.SemaphoreType.DMA((2,)),
            pltpu.SemaphoreType.DMA((4,)),
            pltpu.SemaphoreType.DMA((4,)),
            pltpu.SemaphoreType.DMA((4,)),
            pltpu.SemaphoreType.DMA((4,)),
            pltpu.SemaphoreType.DMA((4,)),
            pltpu.SemaphoreType.DMA((4,)),
            pltpu.SemaphoreType.DMA((4,)),
            pltpu.SemaphoreType.DMA((4,)),
        ],
        compiler_params=pltpu.CompilerParams(collective_id=0),
    )(x, Wq, K_ext, V_ext, Wo)
